# baseline (speedup 1.0000x reference)
import jax
import jax.numpy as jnp
from jax import lax
from jax.experimental import pallas as pl
from jax.experimental.pallas import tpu as pltpu
from jax.experimental.pallas import tpu_sc as plsc

_ROW_INDICES = (5, 8, 7, 16, 256, 123)
_NUM_ROWS = len(_ROW_INDICES)
_DIM = 64
_LANES = 16

_BLOCK_STARTS = tuple(sorted({i - i % 8 for i in _ROW_INDICES}))
_ROW_COORDS = tuple(
    (_BLOCK_STARTS.index(i - i % 8), i % 8) for i in _ROW_INDICES
)
_NUM_BLOCKS = len(_BLOCK_STARTS)


def _gather_body(table_hbm, out_hbm, blocks_v, rows_v, sem):
    wid = lax.axis_index("s") * 2 + lax.axis_index("c")

    @pl.when(wid == 0)
    def _():
        for b, start in enumerate(_BLOCK_STARTS):
            pltpu.async_copy(
                table_hbm.at[pl.ds(start, 8)], blocks_v.at[b], sem
            ).wait()
        for out_row, (blk, row) in enumerate(_ROW_COORDS):
            for c in range(_DIM // _LANES):
                sl = pl.ds(c * _LANES, _LANES)
                rows_v[out_row, sl] = blocks_v[blk, row, sl]
        pltpu.sync_copy(rows_v, out_hbm)


def kernel(input):
    mesh = plsc.VectorSubcoreMesh(core_axis_name="c", subcore_axis_name="s")
    gather = pl.kernel(
        _gather_body,
        mesh=mesh,
        out_type=jax.ShapeDtypeStruct((_NUM_ROWS, _DIM), jnp.float32),
        scratch_types=[
            pltpu.VMEM((_NUM_BLOCKS, 8, _DIM), jnp.float32),
            pltpu.VMEM((_NUM_ROWS, _DIM), jnp.float32),
            pltpu.SemaphoreType.DMA,
        ],
    )
    return gather(input)


# speedup vs baseline: 1.0056x; 1.0056x over previous
import jax
import jax.numpy as jnp
from jax import lax
from jax.experimental import pallas as pl
from jax.experimental.pallas import tpu as pltpu
from jax.experimental.pallas import tpu_sc as plsc

_ROW_INDICES = (5, 8, 7, 16, 256, 123)
_NUM_ROWS = len(_ROW_INDICES)
_DIM = 64
_LANES = 16

_BLOCK_STARTS = tuple(sorted({i - i % 8 for i in _ROW_INDICES}))
_ROW_COORDS = tuple(
    (_BLOCK_STARTS.index(i - i % 8), i % 8) for i in _ROW_INDICES
)
_NUM_BLOCKS = len(_BLOCK_STARTS)


def _gather_body(table_hbm, out_hbm, blocks_v, rows_v, sem):
    wid = lax.axis_index("s") * 2 + lax.axis_index("c")

    @pl.when(wid == 0)
    def _():
        for b, start in enumerate(_BLOCK_STARTS):
            pltpu.async_copy(
                table_hbm.at[pl.ds(start, 8)], blocks_v.at[b], sem
            ).wait()
        for out_row, (blk, row) in enumerate(_ROW_COORDS):
            for c in range(_DIM // _LANES):
                sl = pl.ds(c * _LANES, _LANES)
                rows_v[out_row, sl] = blocks_v[blk, row, sl]
        pltpu.sync_copy(rows_v, out_hbm)


def kernel(input):
    mesh = plsc.VectorSubcoreMesh(
        core_axis_name="c", subcore_axis_name="s", num_cores=1
    )
    gather = pl.kernel(
        _gather_body,
        mesh=mesh,
        out_type=jax.ShapeDtypeStruct((_NUM_ROWS, _DIM), jnp.float32),
        scratch_types=[
            pltpu.VMEM((_NUM_BLOCKS, 8, _DIM), jnp.float32),
            pltpu.VMEM((_NUM_ROWS, _DIM), jnp.float32),
            pltpu.SemaphoreType.DMA,
        ],
    )
    return gather(input)


# speedup vs baseline: 16.8240x; 16.7311x over previous
import jax
import jax.numpy as jnp
from jax import lax
from jax.experimental import pallas as pl
from jax.experimental.pallas import tpu as pltpu
from jax.experimental.pallas import tpu_sc as plsc

_ROW_INDICES = (5, 8, 7, 16, 256, 123)
_NUM_ROWS = len(_ROW_INDICES)
_DIM = 64
_LANES = 16

_CHUNK_STARTS = tuple(sorted({i - i % 128 for i in _ROW_INDICES}))
_NUM_CHUNKS = len(_CHUNK_STARTS)
_COORDS = tuple(
    (_CHUNK_STARTS.index(i - i % 128), (i % 128) // _LANES, i % _LANES)
    for i in _ROW_INDICES
)
_WINDOWS = tuple(sorted({(blk, win) for blk, win, _ in _COORDS}))


def _gather_body(tbl_hbm, out_hbm, chunks_v, outp_v, sem):
    wid = lax.axis_index("s") * 2 + lax.axis_index("c")

    @pl.when(wid == 0)
    def _():
        copies = [
            pltpu.async_copy(
                tbl_hbm.at[:, pl.ds(start, 128)], chunks_v.at[k], sem
            )
            for k, start in enumerate(_CHUNK_STARTS)
        ]
        for c in copies:
            c.wait()

        pos = lax.iota(jnp.int32, _LANES)
        gidx = jnp.zeros((_LANES,), jnp.int32)
        for j, (_, _, lane_in_win) in enumerate(_COORDS):
            gidx = jnp.where(pos == j, lane_in_win, gidx)
        dnums = lax.GatherDimensionNumbers(
            offset_dims=(), collapsed_slice_dims=(0,), start_index_map=(0,)
        )

        for c in range(_DIM):
            gathered = {}
            for blk, win in _WINDOWS:
                w = chunks_v[blk, c, pl.ds(win * _LANES, _LANES)]
                gathered[(blk, win)] = lax.gather(
                    w, gidx[:, None], dnums, (1,),
                    mode=lax.GatherScatterMode.PROMISE_IN_BOUNDS,
                )
            res = gathered[_WINDOWS[0]]
            for j, (blk, win, _) in enumerate(_COORDS):
                res = jnp.where(pos == j, gathered[(blk, win)], res)
            outp_v[c, :] = res
        pltpu.sync_copy(outp_v, out_hbm)


def kernel(input):
    tbl = input.T
    mesh = plsc.VectorSubcoreMesh(
        core_axis_name="c", subcore_axis_name="s", num_cores=1
    )
    gather = pl.kernel(
        _gather_body,
        mesh=mesh,
        out_type=jax.ShapeDtypeStruct((_DIM, _LANES), jnp.float32),
        scratch_types=[
            pltpu.VMEM((_NUM_CHUNKS, _DIM, 128), jnp.float32),
            pltpu.VMEM((_DIM, _LANES), jnp.float32),
            pltpu.SemaphoreType.DMA,
        ],
    )
    return gather(tbl)[:, :_NUM_ROWS].T


# speedup vs baseline: 18.0065x; 1.0703x over previous
import jax
import jax.numpy as jnp
from jax import lax
from jax.experimental import pallas as pl
from jax.experimental.pallas import tpu as pltpu
from jax.experimental.pallas import tpu_sc as plsc

_ROW_INDICES = (5, 8, 7, 16, 256, 123)
_NUM_ROWS = len(_ROW_INDICES)
_DIM = 64
_LANES = 16
_SLAB = 8
_NUM_WORKERS = _DIM // _SLAB

_CHUNK_STARTS = tuple(sorted({i - i % 128 for i in _ROW_INDICES}))
_NUM_CHUNKS = len(_CHUNK_STARTS)
_COORDS = tuple(
    (_CHUNK_STARTS.index(i - i % 128), (i % 128) // _LANES, i % _LANES)
    for i in _ROW_INDICES
)
_WINDOWS = tuple(sorted({(blk, win) for blk, win, _ in _COORDS}))


def _gather_body(tbl_hbm, out_hbm, chunks_v, outp_v, sem):
    sid = lax.axis_index("s")

    @pl.when(sid < _NUM_WORKERS)
    def _():
        base = sid * _SLAB
        copies = [
            pltpu.async_copy(
                tbl_hbm.at[pl.ds(base, _SLAB), pl.ds(start, 128)],
                chunks_v.at[k],
                sem,
            )
            for k, start in enumerate(_CHUNK_STARTS)
        ]
        for c in copies:
            c.wait()

        pos = lax.iota(jnp.int32, _LANES)
        gidx = jnp.zeros((_LANES,), jnp.int32)
        for j, (_, _, lane_in_win) in enumerate(_COORDS):
            gidx = jnp.where(pos == j, lane_in_win, gidx)
        dnums = lax.GatherDimensionNumbers(
            offset_dims=(), collapsed_slice_dims=(0,), start_index_map=(0,)
        )

        for c in range(_SLAB):
            gathered = {}
            for blk, win in _WINDOWS:
                w = chunks_v[blk, c, pl.ds(win * _LANES, _LANES)]
                gathered[(blk, win)] = lax.gather(
                    w, gidx[:, None], dnums, (1,),
                    mode=lax.GatherScatterMode.PROMISE_IN_BOUNDS,
                )
            res = gathered[_WINDOWS[0]]
            for j, (blk, win, _) in enumerate(_COORDS):
                res = jnp.where(pos == j, gathered[(blk, win)], res)
            outp_v[c, :] = res
        pltpu.sync_copy(outp_v, out_hbm.at[pl.ds(base, _SLAB)])


def kernel(input):
    tbl = input.T
    mesh = plsc.VectorSubcoreMesh(
        core_axis_name="c", subcore_axis_name="s", num_cores=1
    )
    gather = pl.kernel(
        _gather_body,
        mesh=mesh,
        out_type=jax.ShapeDtypeStruct((_DIM, _LANES), jnp.float32),
        scratch_types=[
            pltpu.VMEM((_NUM_CHUNKS, _SLAB, 128), jnp.float32),
            pltpu.VMEM((_SLAB, _LANES), jnp.float32),
            pltpu.SemaphoreType.DMA,
        ],
    )
    return gather(tbl)[:, :_NUM_ROWS].T


# speedup vs baseline: 18.3152x; 1.0171x over previous
import jax
import jax.numpy as jnp
from jax import lax
from jax.experimental import pallas as pl
from jax.experimental.pallas import tpu as pltpu
from jax.experimental.pallas import tpu_sc as plsc

_ROW_INDICES = (5, 8, 7, 16, 256, 123)
_NUM_ROWS = len(_ROW_INDICES)
_DIM = 64
_LANES = 16

_CHUNK_STARTS = tuple(sorted({i - i % 128 for i in _ROW_INDICES}))
_NUM_CHUNKS = len(_CHUNK_STARTS)
_COORDS = tuple(
    (_CHUNK_STARTS.index(i - i % 128), (i % 128) // _LANES, i % _LANES)
    for i in _ROW_INDICES
)
_WINDOWS = tuple(sorted({(blk, win) for blk, win, _ in _COORDS}))


def _gather_body(tbl_hbm, out_hbm, chunks_v, rows_v, sem):
    sid = lax.axis_index("s")

    @pl.when(sid == 0)
    def _():
        copies = [
            pltpu.async_copy(
                tbl_hbm.at[:, pl.ds(start, 128)], chunks_v.at[k], sem
            )
            for k, start in enumerate(_CHUNK_STARTS)
        ]
        for c in copies:
            c.wait()

        pos = lax.iota(jnp.int32, _LANES)
        for q in range(_DIM // _LANES):
            acc = [jnp.zeros((_LANES,), jnp.float32) for _ in _ROW_INDICES]
            for t in range(_LANES):
                c = q * _LANES + t
                wvals = {
                    (blk, win): chunks_v[blk, c, pl.ds(win * _LANES, _LANES)]
                    for blk, win in _WINDOWS
                }
                for j, (blk, win, lane) in enumerate(_COORDS):
                    acc[j] = jnp.where(pos == t, wvals[(blk, win)][lane], acc[j])
            for j in range(_NUM_ROWS):
                rows_v[j, pl.ds(q * _LANES, _LANES)] = acc[j]
        pltpu.sync_copy(rows_v, out_hbm)


def kernel(input):
    tbl = input.T
    mesh = plsc.VectorSubcoreMesh(
        core_axis_name="c", subcore_axis_name="s", num_cores=1
    )
    gather = pl.kernel(
        _gather_body,
        mesh=mesh,
        out_type=jax.ShapeDtypeStruct((_NUM_ROWS, _DIM), jnp.float32),
        scratch_types=[
            pltpu.VMEM((_NUM_CHUNKS, _DIM, 128), jnp.float32),
            pltpu.VMEM((_NUM_ROWS, _DIM), jnp.float32),
            pltpu.SemaphoreType.DMA,
        ],
    )
    return gather(tbl)
